# baseline (speedup 1.0000x reference)
import jax
import jax.numpy as jnp
from jax.experimental import pallas as pl
from jax.experimental.pallas import tpu as pltpu

CLS_TOKEN = 1.0
SEP_TOKEN = 2.0


def _body(x_ref, pos_ref, seg_ref, w_ref, b_ref, out_ref, posm, crow, erow):
    i = pl.program_id(0)
    wb = w_ref[:].astype(jnp.bfloat16)

    @pl.when(i == 0)
    def _once():
        seg0 = seg_ref[0:1, :]
        n = pos_ref.shape[0]
        posm[:] = pos_ref[1:n - 1, :] + seg0
        edge_in = jnp.concatenate(
            [pos_ref[0:1, :] + (CLS_TOKEN), pos_ref[n - 1:n, :] + (SEP_TOKEN)],
            axis=0) + seg0
        eout = jnp.dot(edge_in.astype(jnp.bfloat16), wb,
                       preferred_element_type=jnp.float32) + b_ref[:]
        crow[:] = eout[0:1]
        erow[:] = eout[1:2]

    vis = (x_ref[0] + posm[:]).astype(jnp.bfloat16)
    acc = jnp.dot(vis, wb, preferred_element_type=jnp.float32)
    n_rows = out_ref.shape[1]
    out_ref[0, 0:1] = crow[:]
    out_ref[0, 1:n_rows - 1] = acc + b_ref[:]
    out_ref[0, n_rows - 1:n_rows] = erow[:]


@jax.jit
def kernel(x, pos_table, seg_table, W, b):
    batch, sig_len, hid = x.shape
    emb = W.shape[1]
    n_rows = sig_len + 2
    b2 = b.reshape(1, emb)
    out = pl.pallas_call(
        _body,
        grid=(batch,),
        in_specs=[
            pl.BlockSpec((1, sig_len, hid), lambda i: (i, 0, 0)),
            pl.BlockSpec((n_rows, hid), lambda i: (0, 0)),
            pl.BlockSpec((2, hid), lambda i: (0, 0)),
            pl.BlockSpec((hid, emb), lambda i: (0, 0)),
            pl.BlockSpec((1, emb), lambda i: (0, 0)),
        ],
        out_specs=pl.BlockSpec((1, n_rows, emb), lambda i: (i, 0, 0)),
        out_shape=jax.ShapeDtypeStruct((batch, n_rows, emb), jnp.float32),
        scratch_shapes=[
            pltpu.VMEM((sig_len, hid), jnp.float32),
            pltpu.VMEM((1, emb), jnp.float32),
            pltpu.VMEM((1, emb), jnp.float32),
        ],
        compiler_params=pltpu.CompilerParams(
            vmem_limit_bytes=110 * 1024 * 1024),
    )(x, pos_table, seg_table, W, b2)
    return out


# speedup vs baseline: 1.0385x; 1.0385x over previous
import jax
import jax.numpy as jnp
from jax.experimental import pallas as pl
from jax.experimental.pallas import tpu as pltpu

CLS_TOKEN = 1.0
SEP_TOKEN = 2.0

def _body(x_ref, pos_ref, seg_ref, w_ref, b_ref, out_ref):
    seg0 = seg_ref[0:1, :]
    h = x_ref.shape[-1]
    cls_row = jnp.full((1, h), CLS_TOKEN, dtype=jnp.float32)
    sep_row = jnp.full((1, h), SEP_TOKEN, dtype=jnp.float32)
    tokens = jnp.concatenate([cls_row, x_ref[0], sep_row], axis=0)
    vis = tokens + pos_ref[:] + seg0
    acc = jnp.dot(vis.astype(jnp.bfloat16), w_ref[:].astype(jnp.bfloat16),
                  preferred_element_type=jnp.float32)
    out_ref[0] = acc + b_ref[:]


@jax.jit
def kernel(x, pos_table, seg_table, W, b):
    batch, sig_len, hid = x.shape
    emb = W.shape[1]
    n_rows = sig_len + 2
    b2 = b.reshape(1, emb)
    out = pl.pallas_call(
        _body,
        grid=(batch,),
        in_specs=[
            pl.BlockSpec((1, sig_len, hid), lambda i: (i, 0, 0)),
            pl.BlockSpec((n_rows, hid), lambda i: (0, 0)),
            pl.BlockSpec((2, hid), lambda i: (0, 0)),
            pl.BlockSpec((hid, emb), lambda i: (0, 0)),
            pl.BlockSpec((1, emb), lambda i: (0, 0)),
        ],
        out_specs=pl.BlockSpec((1, n_rows, emb), lambda i: (i, 0, 0)),
        out_shape=jax.ShapeDtypeStruct((batch, n_rows, emb), jnp.float32),
    )(x, pos_table, seg_table, W, b2)
    return out
